# baseline (speedup 1.0000x reference)
import jax
import jax.numpy as jnp
from jax.experimental import pallas as pl
from jax.experimental.pallas import tpu as pltpu

ALPHA = 0.2
BLK = 512
BF16 = jnp.bfloat16
F32 = jnp.float32


def _elu(v):
    return jnp.where(v > 0.0, v, jnp.exp(v) - 1.0)


def _gat_body(adj_ref, x_ref, wcat_ref, a1_ref, a2_ref, r8_ref,
              gcw_ref, gcb_ref, wout_ref, encw_ref, encb_ref, ao1_ref, ao2_ref,
              y_ref, z_ref, xo_ref,
              adj16_s, eb_s, cbt_s, v_s, h1_s, hg_s, h2_s,
              ebo_s, cbo_s, cbot_s, vo_s):
    p = pl.program_id(0)
    i = pl.program_id(1)
    nheads = eb_s.shape[1]
    nh = v_s.shape[1] - nheads
    nhid = nh // nheads
    nclass = wout_ref.shape[1]
    w = nclass + 1
    beta = 1.0 - ALPHA
    r = pl.ds(i * BLK, BLK)

    @pl.when((p == 0) & (i == 0))
    def _prologue_a():
        h = jnp.dot(x_ref[...], wcat_ref[...], preferred_element_type=F32)
        f = jnp.dot(h, a1_ref[...], preferred_element_type=F32)
        g = jnp.dot(h, a2_ref[...], preferred_element_type=F32)
        eb_s[...] = jnp.exp(-beta * f)
        cbt_s[...] = jnp.transpose(jnp.exp(-beta * g))
        ega = jnp.exp(-ALPHA * g)
        v_s[:, :nh] = (jnp.dot(ega, r8_ref[...],
                               preferred_element_type=F32) * h).astype(BF16)
        v_s[:, nh:] = ega.astype(BF16)

    @pl.when(p == 0)
    def _phase0():
        adjh = adj_ref[...].astype(BF16)
        adj16_s[r, :] = adjh
        v16 = v_s[...]
        ebr = eb_s[r, :]
        one = jnp.ones((), F32)
        for t in range(nheads):
            q = ebr[:, t:t + 1] * cbt_s[t:t + 1, :]
            e1 = jnp.minimum(q, one).astype(BF16) * adjh
            nv = jnp.dot(e1, v16, preferred_element_type=F32)
            lo, hi = t * nhid, (t + 1) * nhid
            h1_s[r, lo:hi] = _elu(nv[:, lo:hi] / nv[:, nh + t:nh + t + 1])
        y_ref[0] = jnp.dot(h1_s[r, :], encw_ref[...],
                           preferred_element_type=F32) + encb_ref[...]

    @pl.when((p == 1) & (i == 0))
    def _prologue_b():
        hg_s[...] = jnp.dot(h1_s[...], gcw_ref[...],
                            preferred_element_type=F32).astype(BF16)

    @pl.when(p == 1)
    def _phase1():
        h2 = jnp.dot(adj16_s[r, :], hg_s[...],
                     preferred_element_type=F32) + gcb_ref[...]
        h2_s[r, :] = h2
        z_ref[0] = jnp.dot(h2, encw_ref[...],
                           preferred_element_type=F32) + encb_ref[...]
        ho = jnp.dot(h2, wout_ref[...], preferred_element_type=F32)
        fo = jnp.dot(ho, ao1_ref[...], preferred_element_type=F32)
        go = jnp.dot(ho, ao2_ref[...], preferred_element_type=F32)
        ebo_s[r, :] = jnp.exp(-beta * fo)
        cbo_s[r, :] = jnp.exp(-beta * go)
        egao = jnp.exp(-ALPHA * go)
        vo_s[r, :nclass] = (egao * ho).astype(BF16)
        vo_s[r, nclass:] = egao.astype(BF16)

    @pl.when((p == 2) & (i == 0))
    def _transpose_go():
        cbot_s[...] = jnp.transpose(cbo_s[...])

    @pl.when(p == 2)
    def _phase2():
        adjh = adj16_s[r, :]
        q = ebo_s[r, :] * cbot_s[...]
        e1 = jnp.minimum(q, jnp.ones((), F32)).astype(BF16) * adjh
        nv = jnp.dot(e1, vo_s[...], preferred_element_type=F32)
        xo = _elu(nv[:, :nclass] / nv[:, nclass:])
        mx = jnp.max(xo, axis=1, keepdims=True)
        lse = mx + jnp.log(jnp.sum(jnp.exp(xo - mx), axis=1, keepdims=True))
        xo_ref[0] = xo - lse


def kernel(x, adj, W_heads, a_heads, W_out, a_out, gc_weight, gc_bias, enc_W, enc_b):
    n, nfeat = x.shape
    nheads, _, nhid = W_heads.shape
    nh = nheads * nhid
    nclass = W_out.shape[1]
    nstruc = enc_W.shape[1]
    wo = nclass + 1

    wcat = jnp.transpose(W_heads, (1, 0, 2)).reshape(nfeat, nh)
    a1 = a_heads[:, 0, :nhid]
    a2 = a_heads[:, 0, nhid:]
    eye = jnp.eye(nheads, dtype=F32)
    A1 = (a1[:, :, None] * eye[:, None, :]).reshape(nh, nheads)
    A2 = (a2[:, :, None] * eye[:, None, :]).reshape(nh, nheads)
    ao1 = a_out[0, :nclass].reshape(nclass, 1)
    ao2 = a_out[0, nclass:].reshape(nclass, 1)
    gcb = gc_bias.reshape(1, nh)
    encb = enc_b.reshape(1, nstruc)
    R8 = jnp.repeat(eye, nhid, axis=1)

    nblk = n // BLK
    fixed = lambda rr, cc: pl.BlockSpec((rr, cc), lambda p, i: (0, 0))

    y3, z3, xo3 = pl.pallas_call(
        _gat_body,
        grid=(3, nblk),
        in_specs=[pl.BlockSpec((BLK, n), lambda p, i: (i * (p == 0), 0)),
                  fixed(n, nfeat),
                  fixed(nfeat, nh),
                  fixed(nh, nheads),
                  fixed(nh, nheads),
                  fixed(nheads, nh),
                  fixed(nh, nh),
                  fixed(1, nh),
                  fixed(nh, nclass),
                  fixed(nh, nstruc),
                  fixed(1, nstruc),
                  fixed(nclass, 1),
                  fixed(nclass, 1)],
        out_specs=[pl.BlockSpec((1, BLK, nstruc), lambda p, i: (p, i, 0)),
                   pl.BlockSpec((1, BLK, nstruc), lambda p, i: (p, i, 0)),
                   pl.BlockSpec((1, BLK, nclass), lambda p, i: (p, i, 0))],
        out_shape=[jax.ShapeDtypeStruct((3, n, nstruc), F32),
                   jax.ShapeDtypeStruct((3, n, nstruc), F32),
                   jax.ShapeDtypeStruct((3, n, nclass), F32)],
        scratch_shapes=[pltpu.VMEM((n, n), BF16),
                        pltpu.VMEM((n, nheads), F32),
                        pltpu.VMEM((nheads, n), F32),
                        pltpu.VMEM((n, nh + nheads), BF16),
                        pltpu.VMEM((n, nh), F32),
                        pltpu.VMEM((n, nh), BF16),
                        pltpu.VMEM((n, nh), F32),
                        pltpu.VMEM((n, 1), F32),
                        pltpu.VMEM((n, 1), F32),
                        pltpu.VMEM((1, n), F32),
                        pltpu.VMEM((n, wo), BF16)],
    )(adj, x, wcat, A1, A2, R8, gc_weight, gcb, W_out, enc_W, encb, ao1, ao2)

    return (xo3[2], y3[0], z3[1])


# speedup vs baseline: 1.0135x; 1.0135x over previous
import jax
import jax.numpy as jnp
from jax.experimental import pallas as pl
from jax.experimental.pallas import tpu as pltpu

ALPHA = 0.2
BLK = 1024
BF16 = jnp.bfloat16
F32 = jnp.float32


def _elu(v):
    return jnp.where(v > 0.0, v, jnp.exp(v) - 1.0)


def _gat_body(adj_ref, x_ref, wcat_ref, a1_ref, a2_ref, r8_ref,
              gcw_ref, gcb_ref, wout_ref, encw_ref, encb_ref, ao1_ref, ao2_ref,
              y_ref, z_ref, xo_ref,
              adj16_s, eb_s, cbt_s, v_s, h1_s, hg_s, h2_s,
              ebo_s, cbo_s, cbot_s, vo_s):
    p = pl.program_id(0)
    i = pl.program_id(1)
    nheads = eb_s.shape[1]
    nh = v_s.shape[1] - nheads
    nhid = nh // nheads
    nclass = wout_ref.shape[1]
    w = nclass + 1
    beta = 1.0 - ALPHA
    r = pl.ds(i * BLK, BLK)

    @pl.when((p == 0) & (i == 0))
    def _prologue_a():
        h = jnp.dot(x_ref[...], wcat_ref[...], preferred_element_type=F32)
        f = jnp.dot(h, a1_ref[...], preferred_element_type=F32)
        g = jnp.dot(h, a2_ref[...], preferred_element_type=F32)
        eb_s[...] = jnp.exp(-beta * f)
        cbt_s[...] = jnp.transpose(jnp.exp(-beta * g))
        ega = jnp.exp(-ALPHA * g)
        v_s[:, :nh] = (jnp.dot(ega, r8_ref[...],
                               preferred_element_type=F32) * h).astype(BF16)
        v_s[:, nh:] = ega.astype(BF16)

    @pl.when(p == 0)
    def _phase0():
        adjh = adj_ref[...].astype(BF16)
        adj16_s[r, :] = adjh
        v16 = v_s[...]
        ebr = eb_s[r, :]
        one = jnp.ones((), F32)
        for t in range(nheads):
            q = ebr[:, t:t + 1] * cbt_s[t:t + 1, :]
            e1 = jnp.minimum(q, one).astype(BF16) * adjh
            nv = jnp.dot(e1, v16, preferred_element_type=F32)
            lo, hi = t * nhid, (t + 1) * nhid
            h1_s[r, lo:hi] = _elu(nv[:, lo:hi] / nv[:, nh + t:nh + t + 1])
        y_ref[0] = jnp.dot(h1_s[r, :], encw_ref[...],
                           preferred_element_type=F32) + encb_ref[...]

    @pl.when((p == 1) & (i == 0))
    def _prologue_b():
        hg_s[...] = jnp.dot(h1_s[...], gcw_ref[...],
                            preferred_element_type=F32).astype(BF16)

    @pl.when(p == 1)
    def _phase1():
        h2 = jnp.dot(adj16_s[r, :], hg_s[...],
                     preferred_element_type=F32) + gcb_ref[...]
        h2_s[r, :] = h2
        z_ref[0] = jnp.dot(h2, encw_ref[...],
                           preferred_element_type=F32) + encb_ref[...]
        ho = jnp.dot(h2, wout_ref[...], preferred_element_type=F32)
        fo = jnp.dot(ho, ao1_ref[...], preferred_element_type=F32)
        go = jnp.dot(ho, ao2_ref[...], preferred_element_type=F32)
        ebo_s[r, :] = jnp.exp(-beta * fo)
        cbo_s[r, :] = jnp.exp(-beta * go)
        egao = jnp.exp(-ALPHA * go)
        vo_s[r, :nclass] = (egao * ho).astype(BF16)
        vo_s[r, nclass:] = egao.astype(BF16)

    @pl.when((p == 2) & (i == 0))
    def _transpose_go():
        cbot_s[...] = jnp.transpose(cbo_s[...])

    @pl.when(p == 2)
    def _phase2():
        adjh = adj16_s[r, :]
        q = ebo_s[r, :] * cbot_s[...]
        e1 = jnp.minimum(q, jnp.ones((), F32)).astype(BF16) * adjh
        nv = jnp.dot(e1, vo_s[...], preferred_element_type=F32)
        xo = _elu(nv[:, :nclass] / nv[:, nclass:])
        mx = jnp.max(xo, axis=1, keepdims=True)
        lse = mx + jnp.log(jnp.sum(jnp.exp(xo - mx), axis=1, keepdims=True))
        xo_ref[0] = xo - lse


def kernel(x, adj, W_heads, a_heads, W_out, a_out, gc_weight, gc_bias, enc_W, enc_b):
    n, nfeat = x.shape
    nheads, _, nhid = W_heads.shape
    nh = nheads * nhid
    nclass = W_out.shape[1]
    nstruc = enc_W.shape[1]
    wo = nclass + 1

    wcat = jnp.transpose(W_heads, (1, 0, 2)).reshape(nfeat, nh)
    a1 = a_heads[:, 0, :nhid]
    a2 = a_heads[:, 0, nhid:]
    eye = jnp.eye(nheads, dtype=F32)
    A1 = (a1[:, :, None] * eye[:, None, :]).reshape(nh, nheads)
    A2 = (a2[:, :, None] * eye[:, None, :]).reshape(nh, nheads)
    ao1 = a_out[0, :nclass].reshape(nclass, 1)
    ao2 = a_out[0, nclass:].reshape(nclass, 1)
    gcb = gc_bias.reshape(1, nh)
    encb = enc_b.reshape(1, nstruc)
    R8 = jnp.repeat(eye, nhid, axis=1)

    nblk = n // BLK
    fixed = lambda rr, cc: pl.BlockSpec((rr, cc), lambda p, i: (0, 0))

    y3, z3, xo3 = pl.pallas_call(
        _gat_body,
        grid=(3, nblk),
        in_specs=[pl.BlockSpec((BLK, n), lambda p, i: (i * (p == 0), 0)),
                  fixed(n, nfeat),
                  fixed(nfeat, nh),
                  fixed(nh, nheads),
                  fixed(nh, nheads),
                  fixed(nheads, nh),
                  fixed(nh, nh),
                  fixed(1, nh),
                  fixed(nh, nclass),
                  fixed(nh, nstruc),
                  fixed(1, nstruc),
                  fixed(nclass, 1),
                  fixed(nclass, 1)],
        out_specs=[pl.BlockSpec((1, BLK, nstruc), lambda p, i: (p, i, 0)),
                   pl.BlockSpec((1, BLK, nstruc), lambda p, i: (p, i, 0)),
                   pl.BlockSpec((1, BLK, nclass), lambda p, i: (p, i, 0))],
        out_shape=[jax.ShapeDtypeStruct((3, n, nstruc), F32),
                   jax.ShapeDtypeStruct((3, n, nstruc), F32),
                   jax.ShapeDtypeStruct((3, n, nclass), F32)],
        scratch_shapes=[pltpu.VMEM((n, n), BF16),
                        pltpu.VMEM((n, nheads), F32),
                        pltpu.VMEM((nheads, n), F32),
                        pltpu.VMEM((n, nh + nheads), BF16),
                        pltpu.VMEM((n, nh), F32),
                        pltpu.VMEM((n, nh), BF16),
                        pltpu.VMEM((n, nh), F32),
                        pltpu.VMEM((n, 1), F32),
                        pltpu.VMEM((n, 1), F32),
                        pltpu.VMEM((1, n), F32),
                        pltpu.VMEM((n, wo), BF16)],
    )(adj, x, wcat, A1, A2, R8, gc_weight, gcb, W_out, enc_W, encb, ao1, ao2)

    return (xo3[2], y3[0], z3[1])
